# baseline (speedup 1.0000x reference)
import functools
import jax
import jax.numpy as jnp
from jax import lax
from jax.experimental import pallas as pl
from jax.experimental.pallas import tpu as pltpu
from jax.experimental.pallas import tpu_sc as plsc

_B = 2
_N = 8192
_CFG = [
    {"npoint": 2048, "radii": (0.2, 0.8), "nsamples": (16, 32)},
    {"npoint": 512, "radii": (0.8, 1.6), "nsamples": (16, 32)},
]


def _fps_body(npoint, n_rows, x_ref, y_ref, z_ref, o_ref, d_ref):
    B = x_ref.shape[0]
    iota = (jax.lax.broadcasted_iota(jnp.int32, (n_rows, 128), 0) * 128
            + jax.lax.broadcasted_iota(jnp.int32, (n_rows, 128), 1))
    op_rows = o_ref.shape[1]
    oiota = (jax.lax.broadcasted_iota(jnp.int32, (op_rows, 128), 0) * 128
             + jax.lax.broadcasted_iota(jnp.int32, (op_rows, 128), 1))
    d_ref[...] = jnp.full_like(d_ref, 1e10)
    o_ref[...] = jnp.zeros_like(o_ref)
    big = n_rows * 128

    def step(i, fars):
        new_fars = []
        for b in range(B):
            far = fars[b]
            x = x_ref[b]
            y = y_ref[b]
            z = z_ref[b]
            eq = iota == far
            zero = jnp.float32(0.0)
            cx = jnp.sum(jnp.where(eq, x, zero))
            cy = jnp.sum(jnp.where(eq, y, zero))
            cz = jnp.sum(jnp.where(eq, z, zero))
            dx = x - cx
            dy = y - cy
            dz = z - cz
            dist = dx * dx + dy * dy + dz * dz
            d = jnp.minimum(d_ref[b], dist)
            d_ref[b] = d
            o_ref[b] = jnp.where(oiota == i, far, o_ref[b])
            m = jnp.max(d)
            nxt = jnp.min(jnp.where(d == m, iota, big))
            new_fars.append(nxt)
        return tuple(new_fars)

    jax.lax.fori_loop(0, npoint, step, (jnp.int32(0),) * B, unroll=False)


def _fps(xyz, npoint):
    B, N, _ = xyz.shape
    n_rows = N // 128
    planes = xyz.reshape(B, n_rows, 128, 3)
    out = pl.pallas_call(
        functools.partial(_fps_body, npoint, n_rows),
        out_shape=jax.ShapeDtypeStruct((B, npoint // 128, 128), jnp.int32),
        scratch_shapes=[pltpu.VMEM((B, n_rows, 128), jnp.float32)],
    )(planes[..., 0], planes[..., 1], planes[..., 2])
    return out.reshape(B, npoint)


def _gather(pts, idx):
    return jax.vmap(lambda p, i: p[i])(pts, idx)


_NC, _NS = 2, 16
_NW = _NC * _NS


def _sc_gather_body(n_chunks, table_hbm, idx_hbm, out_hbm, idx_v, rows_v, sem):
    wid = lax.axis_index("s") * _NC + lax.axis_index("c")
    pltpu.sync_copy(idx_hbm.at[wid], idx_v)
    cps = []
    for j in range(n_chunks):
        cps.append(pltpu.async_copy(table_hbm.at[idx_v.at[j]], rows_v.at[j], sem))
    for c in cps:
        c.wait()
    pltpu.sync_copy(rows_v, out_hbm.at[wid])


@functools.partial(jax.jit, static_argnums=(2,))
def _sc_gather(table, idx, D):
    TOT = idx.shape[0]
    n_chunks = TOT // (_NW * 128)
    idx3 = idx.reshape(_NW, n_chunks, 128)
    mesh = plsc.VectorSubcoreMesh(core_axis_name="c", subcore_axis_name="s")
    out = pl.kernel(
        functools.partial(_sc_gather_body, n_chunks),
        out_type=jax.ShapeDtypeStruct((_NW, n_chunks, 128, D), jnp.float32),
        mesh=mesh,
        compiler_params=pltpu.CompilerParams(use_tc_tiling_on_sc=False),
        scratch_types=[
            pltpu.VMEM((n_chunks, 128), jnp.int32),
            pltpu.VMEM((n_chunks, 128, D), jnp.float32),
            pltpu.SemaphoreType.DMA,
        ],
    )(table, idx3)
    return out.reshape(TOT, D)


def _sqdist(a, b):
    return (jnp.sum(a * a, -1)[:, :, None]
            - 2.0 * jnp.einsum('bmc,bnc->bmn', a, b)
            + jnp.sum(b * b, -1)[:, None, :])


def _mlp(x, ws):
    for lyr in ws:
        x = jax.nn.relu(x @ lyr["W"] + lyr["b"])
    return x


def _agg_body(x_ref, w_ref, b_ref, o_ref):
    o_ref[...] = jax.nn.relu(
        jnp.dot(x_ref[...], w_ref[...], preferred_element_type=jnp.float32)
        + b_ref[...])


def _agg_matmul(x, W, b):
    B, M, K = x.shape
    Co = W.shape[1]
    x2 = x.reshape(B * M, K)
    out = pl.pallas_call(
        _agg_body,
        out_shape=jax.ShapeDtypeStruct((B * M, Co), jnp.float32),
    )(x2, W, b.reshape(1, Co))
    return out.reshape(B, M, Co)


def _sa_layer(xyz, feats, cfg, p):
    B, N, _ = xyz.shape
    C = feats.shape[-1]
    npoint = cfg["npoint"]
    idx = _fps(xyz, npoint)
    new_xyz = jnp.take_along_axis(xyz, idx[..., None], axis=1)
    d2 = _sqdist(new_xyz, xyz)
    D = 16 if (3 + C) <= 16 else ((3 + C + 15) // 16) * 16
    table = jnp.concatenate(
        [xyz, feats, jnp.zeros((B, N, D - 3 - C), jnp.float32)], axis=-1
    ).reshape(B * N, D)
    boff = (jnp.arange(B, dtype=jnp.int32) * N)[:, None, None]
    negd32, knn32 = jax.lax.top_k(-d2, 32)
    outs = []
    for r, ns, ws in zip(cfg["radii"], cfg["nsamples"], p["scales"]):
        negd, knn = negd32[..., :ns], knn32[..., :ns]
        within = (-negd) <= r * r
        knn = jnp.where(within, knn, knn[..., :1])
        rows = _sc_gather(table, (knn + boff).reshape(-1), D)
        rows = rows.reshape(B, npoint, ns, D)
        gx = rows[..., :3] - new_xyz[:, :, None, :]
        g = jnp.concatenate([gx, rows[..., 3:3 + C]], axis=-1)
        h = _mlp(g, ws)
        outs.append(jnp.max(h, axis=2))
    out = jnp.concatenate(outs, axis=-1)
    out = _agg_matmul(out, p["agg"]["W"], p["agg"]["b"])
    return new_xyz, out


def kernel(points, params):
    xyz = points[:, 1:4].reshape(_B, _N, 3)
    feats = points[:, 4:].reshape(_B, _N, -1)
    for cfg, p in zip(_CFG, params):
        xyz, feats = _sa_layer(xyz, feats, cfg, p)
    return feats


# speedup vs baseline: 3.0505x; 3.0505x over previous
import functools
import jax
import jax.numpy as jnp
from jax import lax
from jax.experimental import pallas as pl
from jax.experimental.pallas import tpu as pltpu
from jax.experimental.pallas import tpu_sc as plsc

_B = 2
_N = 8192
_CFG = [
    {"npoint": 2048, "radii": (0.2, 0.8), "nsamples": (16, 32)},
    {"npoint": 512, "radii": (0.8, 1.6), "nsamples": (16, 32)},
]


def _fps_body(npoint, n_rows, x_ref, y_ref, z_ref, o_ref, d_ref):
    B = x_ref.shape[0]
    iota = (jax.lax.broadcasted_iota(jnp.int32, (n_rows, 128), 0) * 128
            + jax.lax.broadcasted_iota(jnp.int32, (n_rows, 128), 1))
    op_rows = o_ref.shape[1]
    oiota = (jax.lax.broadcasted_iota(jnp.int32, (op_rows, 128), 0) * 128
             + jax.lax.broadcasted_iota(jnp.int32, (op_rows, 128), 1))
    d_ref[...] = jnp.full_like(d_ref, 1e10)
    o_ref[...] = jnp.zeros_like(o_ref)
    big = n_rows * 128

    def step(i, fars):
        new_fars = []
        for b in range(B):
            far = fars[b]
            x = x_ref[b]
            y = y_ref[b]
            z = z_ref[b]
            eq = iota == far
            zero = jnp.float32(0.0)
            cx = jnp.sum(jnp.where(eq, x, zero))
            cy = jnp.sum(jnp.where(eq, y, zero))
            cz = jnp.sum(jnp.where(eq, z, zero))
            dx = x - cx
            dy = y - cy
            dz = z - cz
            dist = dx * dx + dy * dy + dz * dz
            d = jnp.minimum(d_ref[b], dist)
            d_ref[b] = d
            o_ref[b] = jnp.where(oiota == i, far, o_ref[b])
            m = jnp.max(d)
            nxt = jnp.min(jnp.where(d == m, iota, big))
            new_fars.append(nxt)
        return tuple(new_fars)

    jax.lax.fori_loop(0, npoint, step, (jnp.int32(0),) * B, unroll=False)


def _fps(xyz, npoint):
    B, N, _ = xyz.shape
    n_rows = N // 128
    planes = xyz.reshape(B, n_rows, 128, 3)
    out = pl.pallas_call(
        functools.partial(_fps_body, npoint, n_rows),
        out_shape=jax.ShapeDtypeStruct((B, npoint // 128, 128), jnp.int32),
        scratch_shapes=[pltpu.VMEM((B, n_rows, 128), jnp.float32)],
    )(planes[..., 0], planes[..., 1], planes[..., 2])
    return out.reshape(B, npoint)


def _gather(pts, idx):
    return jax.vmap(lambda p, i: p[i])(pts, idx)


_NC, _NS = 2, 16
_NW = _NC * _NS


def _sc_gather_body(n_chunks, table_hbm, idx_hbm, out_hbm, idx_v, rows_v, sem):
    wid = lax.axis_index("s") * _NC + lax.axis_index("c")
    pltpu.sync_copy(idx_hbm.at[wid], idx_v)
    cps = []
    for j in range(n_chunks):
        cps.append(pltpu.async_copy(table_hbm.at[idx_v.at[j]], rows_v.at[j], sem))
    for c in cps:
        c.wait()
    pltpu.sync_copy(rows_v, out_hbm.at[wid])


@functools.partial(jax.jit, static_argnums=(2,))
def _sc_gather(table, idx, D):
    TOT = idx.shape[0]
    n_chunks = TOT // (_NW * 128)
    idx3 = idx.reshape(_NW, n_chunks, 128)
    mesh = plsc.VectorSubcoreMesh(core_axis_name="c", subcore_axis_name="s")
    out = pl.kernel(
        functools.partial(_sc_gather_body, n_chunks),
        out_type=jax.ShapeDtypeStruct((_NW, n_chunks, 128, D), jnp.float32),
        mesh=mesh,
        compiler_params=pltpu.CompilerParams(use_tc_tiling_on_sc=False),
        scratch_types=[
            pltpu.VMEM((n_chunks, 128), jnp.int32),
            pltpu.VMEM((n_chunks, 128, D), jnp.float32),
            pltpu.SemaphoreType.DMA,
        ],
    )(table, idx3)
    return out.reshape(TOT, D)


def _sqdist(a, b):
    return (jnp.sum(a * a, -1)[:, :, None]
            - 2.0 * jnp.einsum('bmc,bnc->bmn', a, b)
            + jnp.sum(b * b, -1)[:, None, :])


_SEL_BLK = 128
_SEL_K = 32


def _sel_body(r2, n, d2_ref, o_v_ref, o_i_ref, o_n_ref, work_ref):
    d2 = d2_ref[0]
    inr = d2 <= jnp.float32(r2)
    inf = jnp.float32(jnp.inf)
    work_ref[...] = jnp.where(inr, d2, inf)
    lane = jax.lax.broadcasted_iota(jnp.int32, (_SEL_BLK, n), 1)
    slot = jax.lax.broadcasted_iota(jnp.int32, (_SEL_BLK, _SEL_K), 1)
    m0 = jnp.min(d2, axis=1, keepdims=True)
    i0 = jnp.min(jnp.where(d2 == m0, lane, n), axis=1, keepdims=True)

    def body(t, c):
        vals, idxs = c
        w = work_ref[...]
        m = jnp.min(w, axis=1, keepdims=True)
        eq = w == m
        ii = jnp.min(jnp.where(eq, lane, n), axis=1, keepdims=True)
        work_ref[...] = jnp.where(lane == ii, inf, w)
        sel = slot == t
        vals = jnp.where(sel, m, vals)
        idxs = jnp.where(sel, ii, idxs)
        return vals, idxs

    init = (jnp.zeros((_SEL_BLK, _SEL_K), jnp.float32),
            jnp.zeros((_SEL_BLK, _SEL_K), jnp.int32))
    vals, idxs = jax.lax.fori_loop(0, _SEL_K, body, init)
    valid = vals < inf
    o_v_ref[0] = jnp.where(valid, vals, m0)
    o_i_ref[0] = jnp.where(valid, idxs, i0)
    o_n_ref[0] = jnp.broadcast_to(i0, (_SEL_BLK, 8))


def _sel_xla(d2, r2):
    B, M, N = d2.shape
    inf = jnp.float32(jnp.inf)
    lane = jnp.arange(N, dtype=jnp.int32)[None, None, :]
    slot = jnp.arange(_SEL_K, dtype=jnp.int32)[None, None, :]
    work0 = jnp.where(d2 <= jnp.float32(r2), d2, inf)

    def body(t, c):
        work, vals, idxs = c
        m = jnp.min(work, axis=-1, keepdims=True)
        eq = work == m
        ii = jnp.min(jnp.where(eq, lane, N), axis=-1, keepdims=True)
        work = jnp.where(lane == ii, inf, work)
        sel = slot == t
        vals = jnp.where(sel, m, vals)
        idxs = jnp.where(sel, ii, idxs)
        return work, vals, idxs

    init = (work0, jnp.zeros((B, M, _SEL_K), jnp.float32),
            jnp.zeros((B, M, _SEL_K), jnp.int32))
    _, vals, idxs = jax.lax.fori_loop(0, _SEL_K, body, init)
    valid = vals < inf
    return (jnp.where(valid, vals, vals[..., 0:1]),
            jnp.where(valid, idxs, idxs[..., 0:1]))


def _sel(d2, r2):
    B, M, N = d2.shape
    return pl.pallas_call(
        functools.partial(_sel_body, float(r2), N),
        grid=(B, M // _SEL_BLK),
        in_specs=[
            pl.BlockSpec((1, _SEL_BLK, N), lambda b, i: (b, i, 0)),
        ],
        out_specs=[
            pl.BlockSpec((1, _SEL_BLK, _SEL_K), lambda b, i: (b, i, 0)),
            pl.BlockSpec((1, _SEL_BLK, _SEL_K), lambda b, i: (b, i, 0)),
            pl.BlockSpec((1, _SEL_BLK, 8), lambda b, i: (b, i, 0)),
        ],
        out_shape=[
            jax.ShapeDtypeStruct((B, M, _SEL_K), jnp.float32),
            jax.ShapeDtypeStruct((B, M, _SEL_K), jnp.int32),
            jax.ShapeDtypeStruct((B, M, 8), jnp.int32),
        ],
        scratch_shapes=[pltpu.VMEM((_SEL_BLK, N), jnp.float32)],
    )(d2)


def _mlp(x, ws):
    for lyr in ws:
        x = jax.nn.relu(x @ lyr["W"] + lyr["b"])
    return x


def _agg_body(x_ref, w_ref, b_ref, o_ref):
    o_ref[...] = jax.nn.relu(
        jnp.dot(x_ref[...], w_ref[...], preferred_element_type=jnp.float32)
        + b_ref[...])


def _agg_matmul(x, W, b):
    B, M, K = x.shape
    Co = W.shape[1]
    x2 = x.reshape(B * M, K)
    out = pl.pallas_call(
        _agg_body,
        out_shape=jax.ShapeDtypeStruct((B * M, Co), jnp.float32),
    )(x2, W, b.reshape(1, Co))
    return out.reshape(B, M, Co)


def _sa_layer(xyz, feats, cfg, p):
    B, N, _ = xyz.shape
    C = feats.shape[-1]
    npoint = cfg["npoint"]
    idx = _fps(xyz, npoint)
    new_xyz = jnp.take_along_axis(xyz, idx[..., None], axis=1)
    D = 16 if (3 + C) <= 16 else ((3 + C + 15) // 16) * 16
    table = jnp.concatenate(
        [xyz, feats, jnp.zeros((B, N, D - 3 - C), jnp.float32)], axis=-1
    ).reshape(B * N, D)
    boff = (jnp.arange(B, dtype=jnp.int32) * N)[:, None, None]
    r_max = max(cfg["radii"])
    vals32, idx32, n0pad = _sel(_sqdist(new_xyz, xyz), r_max * r_max)
    n0 = n0pad[..., :1]
    outs = []
    for r, ns, ws in zip(cfg["radii"], cfg["nsamples"], p["scales"]):
        within = vals32[..., :ns] <= r * r
        knn = jnp.where(within, idx32[..., :ns], n0)
        rows = _sc_gather(table, (knn + boff).reshape(-1), D)
        rows = rows.reshape(B, npoint, ns, D)
        gx = rows[..., :3] - new_xyz[:, :, None, :]
        g = jnp.concatenate([gx, rows[..., 3:3 + C]], axis=-1)
        h = _mlp(g, ws)
        outs.append(jnp.max(h, axis=2))
    out = jnp.concatenate(outs, axis=-1)
    out = _agg_matmul(out, p["agg"]["W"], p["agg"]["b"])
    return new_xyz, out


def kernel(points, params):
    xyz = points[:, 1:4].reshape(_B, _N, 3)
    feats = points[:, 4:].reshape(_B, _N, -1)
    for cfg, p in zip(_CFG, params):
        xyz, feats = _sa_layer(xyz, feats, cfg, p)
    return feats
